# baseline (speedup 1.0000x reference)
import functools

import jax
import jax.numpy as jnp
from jax.experimental import pallas as pl

N = 256
F = 64
H = 4
NRBF = 50
BLK = 64


def _silu(x):
    return x * jax.nn.sigmoid(x)


def _bmm(w, x):
    wb = jnp.broadcast_to(w[None], (x.shape[0],) + w.shape)
    return jax.lax.dot_general(
        wb, x, (((2,), (1,)), ((0,), (0,))),
        preferred_element_type=jnp.float32)


def _sake_block_kernel(
    h_ref, ht_ref, x_ref, xt_ref, v_ref,
    win_jt_ref, win_i_ref, bin_ref, means_ref, betas_ref,
    w0jt_ref, w0i_ref, w0xt_ref, b0_ref,
    w1t_ref, b1_ref, wsemt_ref, bsem_ref, gamma_ref,
    wxt_ref, wp0_ref, bp0_ref, wp1_ref, bp1_ref, vmix_ref,
    wn0h_ref, wn0he_ref, wn0hc_ref, bn0_ref, wn1_ref, bn1_ref,
    wv0_ref, bv0_ref, wv1_ref,
    hn_ref, xn_ref, vn_ref,
):
    B = BLK
    i0 = pl.program_id(0) * B
    f32 = jnp.float32

    ht_all = ht_ref[...]
    h_blk = h_ref[pl.ds(i0, B), :]
    x_blk = x_ref[pl.ds(i0, B), :]
    v_blk = v_ref[...]

    dx = [xt_ref[d:d + 1, :] - x_ref[pl.ds(i0, B), d:d + 1] for d in range(3)]
    d2 = dx[0] * dx[0] + dx[1] * dx[1] + dx[2] * dx[2]
    dist = jnp.sqrt(d2 + 1e-10)

    a_jt = jnp.dot(win_jt_ref[...], ht_all,
                   preferred_element_type=f32)
    b_i = (jnp.dot(h_blk, win_i_ref[...], preferred_element_type=f32)
           + bin_ref[...])
    e1t = a_jt[None, :, :] + b_i[:, :, None]

    cut = 0.5 * (jnp.cos(dist * (jnp.pi / 5.0)) + 1.0)
    cut = cut * (dist < 5.0).astype(f32)
    ed = jnp.exp(-dist)
    t = ed[:, None, :] - means_ref[...][None, :, :]
    rbft = jnp.exp(-betas_ref[...][None, :, :] * t * t)
    xfeatt = rbft * (e1t * cut[:, None, :])

    xfa = jnp.concatenate([xfeatt, dist[:, None, :]], axis=1)
    c_jt = (jnp.dot(w0jt_ref[...], ht_all, preferred_element_type=f32)
            + b0_ref[...])
    d_i = jnp.dot(h_blk, w0i_ref[...], preferred_element_type=f32)
    pre0 = (_bmm(w0xt_ref[...], xfa)
            + c_jt[None, :, :]
            + d_i[:, :, None])
    yt = _silu(pre0)
    hemt = _bmm(w1t_ref[...], yt) + b1_ref[...][None, :, :]

    cols = jax.lax.broadcasted_iota(jnp.int32, (B, N), 1)
    rows = jax.lax.broadcasted_iota(jnp.int32, (B, N), 0) + i0
    eye = (cols == rows).astype(f32)
    neg_masked_dist = -(dist + 1e5 * eye)

    semt = _bmm(wsemt_ref[...], hemt) + bsem_ref[...][None, :, :]

    combs = []
    for hd in range(H):
        s = semt[:, hd, :]
        s = jnp.where(s > 0, s, 2.0 * (jnp.exp(s * 0.5) - 1.0))
        z = s - 1e5 * eye + gamma_ref[0, hd] * neg_masked_dist
        z = z - jnp.max(z, axis=1, keepdims=True)
        z = jnp.exp(z)
        combs.append(z / jnp.sum(z, axis=1, keepdims=True))

    scat = jnp.concatenate([hemt * c[:, None, :] for c in combs],
                           axis=1)
    acct = _bmm(wxt_ref[...], scat)
    he_contrib = jnp.dot(jnp.sum(scat, axis=2), wn0he_ref[...],
                         preferred_element_type=f32)
    coefft = jnp.tanh(acct)

    inv = 1.0 / (dist + 1e-5)
    cs = [jnp.sum(coefft * (dx[d] * inv)[:, None, :], axis=2) * (1.0 / N)
          for d in range(3)]

    cnorm = cs[0] * cs[0] + cs[1] * cs[1] + cs[2] * cs[2]
    hcomb = _silu(jnp.dot(cnorm, wp0_ref[...], preferred_element_type=f32)
                  + bp0_ref[...])
    hcomb = _silu(jnp.dot(hcomb, wp1_ref[...], preferred_element_type=f32)
                  + bp1_ref[...])

    dv = jnp.concatenate(
        [jnp.dot(cs[d], vmix_ref[...], preferred_element_type=f32)
         for d in range(3)], axis=1)

    pre = (jnp.dot(h_blk, wn0h_ref[...], preferred_element_type=f32)
           + he_contrib
           + jnp.dot(hcomb, wn0hc_ref[...], preferred_element_type=f32)
           + bn0_ref[...])
    o = _silu(pre)
    o = _silu(jnp.dot(o, wn1_ref[...], preferred_element_type=f32)
              + bn1_ref[...])
    h_new = h_blk + o

    sc = _silu(jnp.dot(h_new, wv0_ref[...], preferred_element_type=f32)
               + bv0_ref[...])
    sc = 2.0 * jax.nn.sigmoid(jnp.dot(sc, wv1_ref[...],
                                      preferred_element_type=f32))
    v_new = dv + sc * v_blk
    x_new = x_blk + v_new

    hn_ref[...] = h_new
    xn_ref[...] = x_new
    vn_ref[...] = v_new


@jax.jit
def kernel(h, x, v, params):
    ep = params["edge_model"]
    win = ep["mlp_in"]["w"]
    w0 = ep["mlp_out0"]["w"]
    wn0 = params["node_mlp0"]["w"]
    wx = params["x_mixing"]["w"]

    def row(b):
        return b.reshape(1, -1)

    def col(b):
        return b.reshape(-1, 1)

    wxt = wx.reshape(F, H, H * F).transpose(2, 1, 0).reshape(H * F, H * F)

    ins = [
        h, h.T, x, x.T, v,
        win[:F].T, win[F:], row(ep["mlp_in"]["b"]),
        col(ep["kernel"]["means"]), col(ep["kernel"]["betas"]),
        w0[:F].T, w0[F:2 * F], w0[2 * F:2 * F + NRBF + 1].T,
        col(ep["mlp_out0"]["b"]),
        ep["mlp_out1"]["w"].T, col(ep["mlp_out1"]["b"]),
        params["semantic_attention_mlp"]["w"].T,
        col(params["semantic_attention_mlp"]["b"]),
        row(jnp.exp(params["log_gamma"])),
        wxt,
        params["post_norm_mlp0"]["w"], row(params["post_norm_mlp0"]["b"]),
        params["post_norm_mlp1"]["w"], row(params["post_norm_mlp1"]["b"]),
        params["v_mixing"]["w"],
        wn0[:F],
        wn0[F:F + H * F].reshape(F, H, F).transpose(1, 0, 2).reshape(H * F, F),
        wn0[F + H * F:],
        row(params["node_mlp0"]["b"]),
        params["node_mlp1"]["w"], row(params["node_mlp1"]["b"]),
        params["velocity_mlp0"]["w"], row(params["velocity_mlp0"]["b"]),
        params["velocity_mlp1"]["w"],
    ]

    def full(a):
        return pl.BlockSpec(a.shape, lambda i: (0,) * a.ndim)

    in_specs = [full(a) for a in ins]
    in_specs[4] = pl.BlockSpec((BLK, 3), lambda i: (i, 0))

    out_shape = [
        jax.ShapeDtypeStruct((N, F), jnp.float32),
        jax.ShapeDtypeStruct((N, 3), jnp.float32),
        jax.ShapeDtypeStruct((N, 3), jnp.float32),
    ]
    out_specs = [
        pl.BlockSpec((BLK, F), lambda i: (i, 0)),
        pl.BlockSpec((BLK, 3), lambda i: (i, 0)),
        pl.BlockSpec((BLK, 3), lambda i: (i, 0)),
    ]

    h_new, x_new, v_new = pl.pallas_call(
        _sake_block_kernel,
        grid=(N // BLK,),
        in_specs=in_specs,
        out_specs=out_specs,
        out_shape=out_shape,
    )(*ins)
    return h_new, x_new, v_new


# speedup vs baseline: 1.0011x; 1.0011x over previous
import jax
import jax.numpy as jnp
from jax.experimental import pallas as pl

N = 256
F = 64
H = 4
NRBF = 50
BLK = 64


def _silu(x):
    return x * jax.nn.sigmoid(x)


def _bmm(w, x):
    wb = jnp.broadcast_to(w[None], (x.shape[0],) + w.shape)
    return jax.lax.dot_general(
        wb, x, (((2,), (1,)), ((0,), (0,))),
        preferred_element_type=jnp.float32)


def _sake_block_kernel(
    h_ref, ht_ref, x_ref, xt_ref, v_ref,
    win_jt_ref, win_i_ref, bin_ref, means_ref, betas_ref,
    w0jt_ref, w0i_ref, w0xt_ref, b0_ref,
    w1t_ref, b1_ref, wsemt_ref, bsem_ref, gamma_ref,
    wxt_ref, wp0_ref, bp0_ref, wp1_ref, bp1_ref, vmix_ref,
    wn0h_ref, wn0he_ref, wn0hc_ref, bn0_ref, wn1_ref, bn1_ref,
    wv0_ref, bv0_ref, wv1_ref,
    hn_ref, xn_ref, vn_ref,
):
    B = BLK
    i0 = pl.program_id(0) * B
    f32 = jnp.float32

    ht_all = ht_ref[...]
    h_blk = h_ref[pl.ds(i0, B), :]
    x_blk = x_ref[pl.ds(i0, B), :]
    v_blk = v_ref[...]

    dx = [xt_ref[d:d + 1, :] - x_ref[pl.ds(i0, B), d:d + 1] for d in range(3)]
    d2 = dx[0] * dx[0] + dx[1] * dx[1] + dx[2] * dx[2]
    dist = jnp.sqrt(d2 + 1e-10)

    a_jt = jnp.dot(win_jt_ref[...], ht_all,
                   preferred_element_type=f32)
    b_i = (jnp.dot(h_blk, win_i_ref[...], preferred_element_type=f32)
           + bin_ref[...])
    e1t = a_jt[None, :, :] + b_i[:, :, None]

    cut = 0.5 * (jnp.cos(dist * (jnp.pi / 5.0)) + 1.0)
    cut = cut * (dist < 5.0).astype(f32)
    ed = jnp.exp(-dist)
    t = ed[:, None, :] - means_ref[...][None, :, :]
    rbft = jnp.exp(-betas_ref[...][None, :, :] * t * t)
    xfeatt = rbft * (e1t * cut[:, None, :])

    xfa = jnp.concatenate([xfeatt, dist[:, None, :]], axis=1)
    c_jt = (jnp.dot(w0jt_ref[...], ht_all, preferred_element_type=f32)
            + b0_ref[...])
    d_i = jnp.dot(h_blk, w0i_ref[...], preferred_element_type=f32)
    pre0 = (_bmm(w0xt_ref[...], xfa)
            + c_jt[None, :, :]
            + d_i[:, :, None])
    yt = _silu(pre0)
    hemt = _bmm(w1t_ref[...], yt) + b1_ref[...][None, :, :]

    cols = jax.lax.broadcasted_iota(jnp.int32, (B, N), 1)
    rows = jax.lax.broadcasted_iota(jnp.int32, (B, N), 0) + i0
    eye = (cols == rows).astype(f32)
    neg_masked_dist = -(dist + 1e5 * eye)

    semt = _bmm(wsemt_ref[...], hemt) + bsem_ref[...][None, :, :]

    combs = []
    for hd in range(H):
        s = semt[:, hd, :]
        s = jnp.where(s > 0, s, 2.0 * (jnp.exp(s * 0.5) - 1.0))
        z = s - 1e5 * eye + gamma_ref[0, hd] * neg_masked_dist
        z = z - jnp.max(z, axis=1, keepdims=True)
        z = jnp.exp(z)
        combs.append(z / jnp.sum(z, axis=1, keepdims=True))

    scat = jnp.concatenate([hemt * c[:, None, :] for c in combs],
                           axis=1)
    acct = _bmm(wxt_ref[...], scat)
    he_contrib = jnp.dot(jnp.sum(scat, axis=2), wn0he_ref[...],
                         preferred_element_type=f32)
    coefft = jnp.tanh(acct)

    inv = 1.0 / (dist + 1e-5)
    cs = [jnp.sum(coefft * (dx[d] * inv)[:, None, :], axis=2) * (1.0 / N)
          for d in range(3)]

    cnorm = cs[0] * cs[0] + cs[1] * cs[1] + cs[2] * cs[2]
    hcomb = _silu(jnp.dot(cnorm, wp0_ref[...], preferred_element_type=f32)
                  + bp0_ref[...])
    hcomb = _silu(jnp.dot(hcomb, wp1_ref[...], preferred_element_type=f32)
                  + bp1_ref[...])

    dv = jnp.concatenate(
        [jnp.dot(cs[d], vmix_ref[...], preferred_element_type=f32)
         for d in range(3)], axis=1)

    pre = (jnp.dot(h_blk, wn0h_ref[...], preferred_element_type=f32)
           + he_contrib
           + jnp.dot(hcomb, wn0hc_ref[...], preferred_element_type=f32)
           + bn0_ref[...])
    o = _silu(pre)
    o = _silu(jnp.dot(o, wn1_ref[...], preferred_element_type=f32)
              + bn1_ref[...])
    h_new = h_blk + o

    sc = _silu(jnp.dot(h_new, wv0_ref[...], preferred_element_type=f32)
               + bv0_ref[...])
    sc = 2.0 * jax.nn.sigmoid(jnp.dot(sc, wv1_ref[...],
                                      preferred_element_type=f32))
    v_new = dv + sc * v_blk
    x_new = x_blk + v_new

    hn_ref[...] = h_new
    xn_ref[...] = x_new
    vn_ref[...] = v_new


@jax.jit
def kernel(h, x, v, params):
    ep = params["edge_model"]
    win = ep["mlp_in"]["w"]
    w0 = ep["mlp_out0"]["w"]
    wn0 = params["node_mlp0"]["w"]
    wx = params["x_mixing"]["w"]

    def row(b):
        return b.reshape(1, -1)

    def col(b):
        return b.reshape(-1, 1)

    wxt = wx.reshape(F, H, H * F).transpose(2, 1, 0).reshape(H * F, H * F)

    ins = [
        h, h.T, x, x.T, v,
        win[:F].T, win[F:], row(ep["mlp_in"]["b"]),
        col(ep["kernel"]["means"]), col(ep["kernel"]["betas"]),
        w0[:F].T, w0[F:2 * F], w0[2 * F:2 * F + NRBF + 1].T,
        col(ep["mlp_out0"]["b"]),
        ep["mlp_out1"]["w"].T, col(ep["mlp_out1"]["b"]),
        params["semantic_attention_mlp"]["w"].T,
        col(params["semantic_attention_mlp"]["b"]),
        row(jnp.exp(params["log_gamma"])),
        wxt,
        params["post_norm_mlp0"]["w"], row(params["post_norm_mlp0"]["b"]),
        params["post_norm_mlp1"]["w"], row(params["post_norm_mlp1"]["b"]),
        params["v_mixing"]["w"],
        wn0[:F],
        wn0[F:F + H * F].reshape(F, H, F).transpose(1, 0, 2).reshape(H * F, F),
        wn0[F + H * F:],
        row(params["node_mlp0"]["b"]),
        params["node_mlp1"]["w"], row(params["node_mlp1"]["b"]),
        params["velocity_mlp0"]["w"], row(params["velocity_mlp0"]["b"]),
        params["velocity_mlp1"]["w"],
    ]

    def full(a):
        return pl.BlockSpec(a.shape, lambda i: (0,) * a.ndim)

    in_specs = [full(a) for a in ins]
    in_specs[4] = pl.BlockSpec((BLK, 3), lambda i: (i, 0))

    out_shape = [
        jax.ShapeDtypeStruct((N, F), jnp.float32),
        jax.ShapeDtypeStruct((N, 3), jnp.float32),
        jax.ShapeDtypeStruct((N, 3), jnp.float32),
    ]
    out_specs = [
        pl.BlockSpec((BLK, F), lambda i: (i, 0)),
        pl.BlockSpec((BLK, 3), lambda i: (i, 0)),
        pl.BlockSpec((BLK, 3), lambda i: (i, 0)),
    ]

    h_new, x_new, v_new = pl.pallas_call(
        _sake_block_kernel,
        grid=(N // BLK,),
        in_specs=in_specs,
        out_specs=out_specs,
        out_shape=out_shape,
    )(*ins)
    return h_new, x_new, v_new
